# baseline (speedup 1.0000x reference)
import functools
import jax
import jax.numpy as jnp
from jax import lax
from jax.experimental import pallas as pl
from jax.experimental.pallas import tpu as pltpu
from jax.experimental.pallas import tpu_sc as plsc

T = 2048
D = 1024
E = 8
F = 512
NG = 2
NSHARED = 2
SCALE = 2.5
FS = F * NSHARED

BS = 128
NB = (T * 2 - E) // BS + E
P = NB * BS
RC = 256

NC = 2
NS = 16
NW = NC * NS
TOKW = T // NW
CHUNK = 16


def _lane_cumsum(a):
    cols = [a[:, 0:1]]
    for j in range(1, a.shape[1]):
        cols.append(cols[-1] + a[:, j:j + 1])
    return jnp.concatenate(cols, axis=1)


def _sub_cumsum(a):
    rows = [a[0:1]]
    for j in range(1, a.shape[0]):
        rows.append(rows[-1] + a[j:j + 1])
    return jnp.concatenate(rows, axis=0)


def _router_body(x_ref, gw_ref, eb_ref, meta_ref, be_ref):
    x = x_ref[...]
    logits = jnp.dot(x, gw_ref[...])
    scores = jax.nn.sigmoid(logits)
    sfc = scores + eb_ref[...]

    def top2sum(grp):
        g1 = jnp.max(grp, axis=-1, keepdims=True)
        eq1 = jnp.where(grp == g1, 1.0, 0.0)
        first1 = (eq1 * _lane_cumsum(eq1)) == 1.0
        g2 = jnp.max(jnp.where(first1, -jnp.inf, grp), axis=-1, keepdims=True)
        return g1 + g2

    gs0 = top2sum(sfc[:, :E // NG])
    gs1 = top2sum(sfc[:, E // NG:])
    gsel_f = jnp.where(gs0 >= gs1, 1.0, 0.0)
    gof = (jax.lax.broadcasted_iota(jnp.int32, (T, E), 1) // (E // NG)
           ).astype(jnp.float32)
    emask_f = gsel_f * (1.0 - gof) + (1.0 - gsel_f) * gof

    masked = jnp.where(emask_f > 0.5, sfc, -1e9)
    m1 = jnp.max(masked, axis=-1, keepdims=True)
    e1 = jnp.where(masked == m1, 1.0, 0.0)
    f1 = (e1 * _lane_cumsum(e1)) == 1.0
    masked2 = jnp.where(f1, -jnp.inf, masked)
    m2 = jnp.max(masked2, axis=-1, keepdims=True)
    e2 = jnp.where(masked2 == m2, 1.0, 0.0)
    f2 = (e2 * _lane_cumsum(e2)) == 1.0

    w1 = jnp.sum(jnp.where(f1, scores, 0.0), axis=-1, keepdims=True)
    w2 = jnp.sum(jnp.where(f2, scores, 0.0), axis=-1, keepdims=True)
    inv = SCALE / (w1 + w2 + 1e-20)
    w1s = w1 * inv
    w2s = w2 * inv

    sel = jnp.where(f1, 1.0, 0.0) + jnp.where(f2, 1.0, 0.0)
    tri = jnp.where(
        jax.lax.broadcasted_iota(jnp.int32, (RC, RC), 0)
        > jax.lax.broadcasted_iota(jnp.int32, (RC, RC), 1), 1.0, 0.0)
    ranks = []
    carry = jnp.zeros((1, E), jnp.float32)
    for i in range(T // RC):
        ch = sel[i * RC:(i + 1) * RC]
        rloc = jnp.dot(tri, ch, preferred_element_type=jnp.float32)
        ranks.append(rloc + carry)
        carry = carry + jnp.sum(ch, axis=0, keepdims=True)
    rank = jnp.concatenate(ranks, axis=0)

    counts_row = carry
    nb_row = jnp.floor(counts_row * (1.0 / BS) + (BS - 1) / BS)
    start_row = (_lane_cumsum(nb_row) - nb_row) * BS
    pos = start_row + rank
    pos1 = jnp.sum(jnp.where(f1, pos, 0.0), axis=-1, keepdims=True)
    pos2 = jnp.sum(jnp.where(f2, pos, 0.0), axis=-1, keepdims=True)

    meta4 = jnp.concatenate([pos1, pos2, w1s, w2s], axis=1)
    meta_ref[...] = jnp.concatenate(
        [lax.transpose(meta4, (1, 0)),
         jnp.zeros((4, T), jnp.float32)], axis=0)

    counts_col = lax.dot_general(sel, jnp.ones((T, 1), jnp.float32),
                                 (((0,), (0,)), ((), ())))
    nb_col = jnp.floor(counts_col * (1.0 / BS) + (BS - 1) / BS)
    end_col = _sub_cumsum(nb_col)
    b_io = jax.lax.broadcasted_iota(jnp.int32, (E, 128), 1).astype(jnp.float32)
    cmp = jnp.where(b_io >= end_col, 1.0, 0.0)
    be_row = jnp.minimum(jnp.sum(cmp, axis=0, keepdims=True), E - 1)
    sub0 = jax.lax.broadcasted_iota(jnp.int32, (8, 128), 0) == 0
    be_ref[...] = jnp.where(sub0, be_row, 0.0).astype(jnp.int32)


def _shared_body(x_ref, wsgu_ref, wsdn_ref, sh_ref):
    x = x_ref[...]
    for hh in range(NSHARED):
        g = jnp.dot(x, wsgu_ref[:, hh * F:(hh + 1) * F],
                    preferred_element_type=jnp.float32)
        u = jnp.dot(x, wsgu_ref[:, FS + hh * F:FS + (hh + 1) * F],
                    preferred_element_type=jnp.float32)
        hq = g * jax.nn.sigmoid(g) * u
        y = jnp.dot(hq, wsdn_ref[hh * F:(hh + 1) * F, :],
                    preferred_element_type=jnp.float32)
        if hh == 0:
            sh_ref[...] = y
        else:
            sh_ref[...] += y


def _routed_body(be_ref, xs_ref, sw_ref, wgu_ref, wdn_ref, y_ref):
    del be_ref
    xb = xs_ref[...]
    gu = jnp.dot(xb, wgu_ref[0], preferred_element_type=jnp.float32)
    g = gu[:, :F]
    h = g * jax.nn.sigmoid(g) * gu[:, F:]
    y = jnp.dot(h, wdn_ref[0], preferred_element_type=jnp.float32)
    y_ref[...] = y * sw_ref[...]


def _dispatch_body(meta_hbm, x_hbm, xs_hbm, sw_hbm,
                   xrows, pf, p1i, p2i, wb1, wb2, sem):
    wid = lax.axis_index("s") * NC + lax.axis_index("c")
    base = wid * TOKW

    def chunk(c, _):
        tb = base + c * CHUNK
        pltpu.sync_copy(meta_hbm.at[0, pl.ds(tb, CHUNK)], pf)
        p1i[...] = pf[...].astype(jnp.int32)
        pltpu.sync_copy(meta_hbm.at[1, pl.ds(tb, CHUNK)], pf)
        p2i[...] = pf[...].astype(jnp.int32)
        pltpu.sync_copy(meta_hbm.at[2, pl.ds(tb, CHUNK)], wb1)
        pltpu.sync_copy(meta_hbm.at[3, pl.ds(tb, CHUNK)], wb2)
        pltpu.sync_copy(x_hbm.at[pl.ds(tb, CHUNK)], xrows)
        pltpu.async_copy(xrows, xs_hbm.at[p1i], sem).wait()
        pltpu.async_copy(xrows, xs_hbm.at[p2i], sem).wait()
        pltpu.async_copy(wb1, sw_hbm.at[p1i], sem).wait()
        pltpu.async_copy(wb2, sw_hbm.at[p2i], sem).wait()
        return 0

    lax.fori_loop(0, TOKW // CHUNK, chunk, 0)


def _combine_body(y_hbm, sh_hbm, meta_hbm, out_hbm,
                  r1, r2, shv, ob, pf, p1i, p2i, sem):
    wid = lax.axis_index("s") * NC + lax.axis_index("c")
    base = wid * TOKW

    def chunk(c, _):
        tb = base + c * CHUNK
        pltpu.sync_copy(meta_hbm.at[0, pl.ds(tb, CHUNK)], pf)
        p1i[...] = pf[...].astype(jnp.int32)
        pltpu.sync_copy(meta_hbm.at[1, pl.ds(tb, CHUNK)], pf)
        p2i[...] = pf[...].astype(jnp.int32)
        pltpu.async_copy(y_hbm.at[p1i], r1, sem).wait()
        pltpu.async_copy(y_hbm.at[p2i], r2, sem).wait()
        pltpu.sync_copy(sh_hbm.at[pl.ds(tb, CHUNK)], shv)

        def row(i, _):
            for v in range(D // 16):
                s = pl.ds(v * 16, 16)
                ob[i, s] = r1[i, s] + r2[i, s] + shv[i, s]
            return 0

        lax.fori_loop(0, CHUNK, row, 0)
        pltpu.sync_copy(ob, out_hbm.at[pl.ds(tb, CHUNK)])
        return 0

    lax.fori_loop(0, TOKW // CHUNK, chunk, 0)


def _make_sc_kernels():
    mesh = plsc.VectorSubcoreMesh(core_axis_name="c", subcore_axis_name="s",
                                  num_cores=NC, num_subcores=NS)
    dispatch = pl.kernel(
        _dispatch_body, mesh=mesh,
        out_type=[jax.ShapeDtypeStruct((P, D), jnp.float32),
                  jax.ShapeDtypeStruct((P,), jnp.float32)],
        scratch_types=[
            pltpu.VMEM((CHUNK, D), jnp.float32),
            pltpu.VMEM((CHUNK,), jnp.float32),
            pltpu.VMEM((CHUNK,), jnp.int32),
            pltpu.VMEM((CHUNK,), jnp.int32),
            pltpu.VMEM((CHUNK,), jnp.float32),
            pltpu.VMEM((CHUNK,), jnp.float32),
            pltpu.SemaphoreType.DMA,
        ])
    combine = pl.kernel(
        _combine_body, mesh=mesh,
        out_type=jax.ShapeDtypeStruct((T, D), jnp.float32),
        scratch_types=[
            pltpu.VMEM((CHUNK, D), jnp.float32),
            pltpu.VMEM((CHUNK, D), jnp.float32),
            pltpu.VMEM((CHUNK, D), jnp.float32),
            pltpu.VMEM((CHUNK, D), jnp.float32),
            pltpu.VMEM((CHUNK,), jnp.float32),
            pltpu.VMEM((CHUNK,), jnp.int32),
            pltpu.VMEM((CHUNK,), jnp.int32),
            pltpu.SemaphoreType.DMA,
        ])
    return dispatch, combine


@jax.jit
def kernel(hidden_states, gate_w, e_bias, w_gate_up, w_down, ws_gate_up,
           ws_down):
    x = hidden_states.reshape(T, D)

    meta, bearr = pl.pallas_call(
        _router_body,
        out_shape=[jax.ShapeDtypeStruct((8, T), jnp.float32),
                   jax.ShapeDtypeStruct((8, 128), jnp.int32)],
    )(x, gate_w, e_bias.reshape(1, E))
    be = bearr[0, :NB]

    _dispatch, _combine = _make_sc_kernels()
    xs, sw = _dispatch(meta, x)

    sh = pl.pallas_call(
        _shared_body,
        out_shape=jax.ShapeDtypeStruct((T, D), jnp.float32),
    )(x, ws_gate_up, ws_down)

    y = pl.pallas_call(
        _routed_body,
        grid_spec=pltpu.PrefetchScalarGridSpec(
            num_scalar_prefetch=1,
            grid=(NB,),
            in_specs=[
                pl.BlockSpec((BS, D), lambda b, be: (b, 0)),
                pl.BlockSpec((BS, 1), lambda b, be: (b, 0)),
                pl.BlockSpec((1, D, 2 * F), lambda b, be: (be[b], 0, 0)),
                pl.BlockSpec((1, F, D), lambda b, be: (be[b], 0, 0)),
            ],
            out_specs=pl.BlockSpec((BS, D), lambda b, be: (b, 0)),
        ),
        out_shape=jax.ShapeDtypeStruct((P, D), jnp.float32),
    )(be, xs, sw.reshape(P, 1), w_gate_up, w_down)

    out = _combine(y, sh, meta)
    return out


# speedup vs baseline: 1.0317x; 1.0317x over previous
import functools
import jax
import jax.numpy as jnp
from jax import lax
from jax.experimental import pallas as pl
from jax.experimental.pallas import tpu as pltpu
from jax.experimental.pallas import tpu_sc as plsc

T = 2048
D = 1024
E = 8
F = 512
NG = 2
NSHARED = 2
SCALE = 2.5
FS = F * NSHARED

BS = 128
NB = (T * 2 - E) // BS + E
P = NB * BS
RC = 256

NC = 2
NS = 16
NW = NC * NS
TOKW = T // NW
CHUNK = 16


def _lane_cumsum(a):
    cols = [a[:, 0:1]]
    for j in range(1, a.shape[1]):
        cols.append(cols[-1] + a[:, j:j + 1])
    return jnp.concatenate(cols, axis=1)


def _sub_cumsum(a):
    rows = [a[0:1]]
    for j in range(1, a.shape[0]):
        rows.append(rows[-1] + a[j:j + 1])
    return jnp.concatenate(rows, axis=0)


def _router_body(x_ref, gw_ref, eb_ref, meta_ref, be_ref):
    x = x_ref[...]
    logits = jnp.dot(x, gw_ref[...])
    scores = jax.nn.sigmoid(logits)
    sfc = scores + eb_ref[...]

    def top2sum(grp):
        g1 = jnp.max(grp, axis=-1, keepdims=True)
        eq1 = jnp.where(grp == g1, 1.0, 0.0)
        first1 = (eq1 * _lane_cumsum(eq1)) == 1.0
        g2 = jnp.max(jnp.where(first1, -jnp.inf, grp), axis=-1, keepdims=True)
        return g1 + g2

    gs0 = top2sum(sfc[:, :E // NG])
    gs1 = top2sum(sfc[:, E // NG:])
    gsel_f = jnp.where(gs0 >= gs1, 1.0, 0.0)
    gof = (jax.lax.broadcasted_iota(jnp.int32, (T, E), 1) // (E // NG)
           ).astype(jnp.float32)
    emask_f = gsel_f * (1.0 - gof) + (1.0 - gsel_f) * gof

    masked = jnp.where(emask_f > 0.5, sfc, -1e9)
    m1 = jnp.max(masked, axis=-1, keepdims=True)
    e1 = jnp.where(masked == m1, 1.0, 0.0)
    f1 = (e1 * _lane_cumsum(e1)) == 1.0
    masked2 = jnp.where(f1, -jnp.inf, masked)
    m2 = jnp.max(masked2, axis=-1, keepdims=True)
    e2 = jnp.where(masked2 == m2, 1.0, 0.0)
    f2 = (e2 * _lane_cumsum(e2)) == 1.0

    w1 = jnp.sum(jnp.where(f1, scores, 0.0), axis=-1, keepdims=True)
    w2 = jnp.sum(jnp.where(f2, scores, 0.0), axis=-1, keepdims=True)
    inv = SCALE / (w1 + w2 + 1e-20)
    w1s = w1 * inv
    w2s = w2 * inv

    sel = jnp.where(f1, 1.0, 0.0) + jnp.where(f2, 1.0, 0.0)
    tri = jnp.where(
        jax.lax.broadcasted_iota(jnp.int32, (RC, RC), 0)
        > jax.lax.broadcasted_iota(jnp.int32, (RC, RC), 1), 1.0, 0.0)
    ranks = []
    carry = jnp.zeros((1, E), jnp.float32)
    for i in range(T // RC):
        ch = sel[i * RC:(i + 1) * RC]
        rloc = jnp.dot(tri, ch, preferred_element_type=jnp.float32)
        ranks.append(rloc + carry)
        carry = carry + jnp.sum(ch, axis=0, keepdims=True)
    rank = jnp.concatenate(ranks, axis=0)

    counts_row = carry
    nb_row = jnp.floor(counts_row * (1.0 / BS) + (BS - 1) / BS)
    start_row = (_lane_cumsum(nb_row) - nb_row) * BS
    pos = start_row + rank
    pos1 = jnp.sum(jnp.where(f1, pos, 0.0), axis=-1, keepdims=True)
    pos2 = jnp.sum(jnp.where(f2, pos, 0.0), axis=-1, keepdims=True)

    meta4 = jnp.concatenate([pos1, pos2, w1s, w2s], axis=1)
    meta_ref[...] = jnp.concatenate(
        [lax.transpose(meta4, (1, 0)),
         jnp.zeros((4, T), jnp.float32)], axis=0)

    counts_col = lax.dot_general(sel, jnp.ones((T, 1), jnp.float32),
                                 (((0,), (0,)), ((), ())))
    nb_col = jnp.floor(counts_col * (1.0 / BS) + (BS - 1) / BS)
    end_col = _sub_cumsum(nb_col)
    b_io = jax.lax.broadcasted_iota(jnp.int32, (E, 128), 1).astype(jnp.float32)
    cmp = jnp.where(b_io >= end_col, 1.0, 0.0)
    be_row = jnp.minimum(jnp.sum(cmp, axis=0, keepdims=True), E - 1)
    sub0 = jax.lax.broadcasted_iota(jnp.int32, (8, 128), 0) == 0
    be_ref[...] = jnp.where(sub0, be_row, 0.0).astype(jnp.int32)


def _shared_body(x_ref, wsgu_ref, wsdn_ref, sh_ref):
    x = x_ref[...]
    for hh in range(NSHARED):
        g = jnp.dot(x, wsgu_ref[:, hh * F:(hh + 1) * F],
                    preferred_element_type=jnp.float32)
        u = jnp.dot(x, wsgu_ref[:, FS + hh * F:FS + (hh + 1) * F],
                    preferred_element_type=jnp.float32)
        hq = g * jax.nn.sigmoid(g) * u
        y = jnp.dot(hq, wsdn_ref[hh * F:(hh + 1) * F, :],
                    preferred_element_type=jnp.float32)
        if hh == 0:
            sh_ref[...] = y
        else:
            sh_ref[...] += y


def _routed_body(be_ref, xs_ref, sw_ref, wgu_ref, wdn_ref, y_ref):
    del be_ref
    xb = xs_ref[...]
    gu = jnp.dot(xb, wgu_ref[0], preferred_element_type=jnp.float32)
    g = gu[:, :F]
    h = g * jax.nn.sigmoid(g) * gu[:, F:]
    y = jnp.dot(h, wdn_ref[0], preferred_element_type=jnp.float32)
    y_ref[...] = y * sw_ref[...]


def _dispatch_body(meta_hbm, x_hbm, xs_hbm, sw_hbm,
                   xrows, mrows, p1i, p2i, wb1, wb2, sem):
    wid = lax.axis_index("s") * NC + lax.axis_index("c")
    base = wid * TOKW

    for r in range(4):
        pltpu.sync_copy(meta_hbm.at[r, pl.ds(base, TOKW)], mrows.at[r])
    pltpu.sync_copy(x_hbm.at[pl.ds(base, TOKW)], xrows)
    for k in range(TOKW // CHUNK):
        sl = pl.ds(k * CHUNK, CHUNK)
        p1i[sl] = mrows[0, sl].astype(jnp.int32)
        p2i[sl] = mrows[1, sl].astype(jnp.int32)
        wb1[sl] = mrows[2, sl]
        wb2[sl] = mrows[3, sl]
    a1 = pltpu.async_copy(xrows, xs_hbm.at[p1i], sem)
    a2 = pltpu.async_copy(xrows, xs_hbm.at[p2i], sem)
    a3 = pltpu.async_copy(wb1, sw_hbm.at[p1i], sem)
    a4 = pltpu.async_copy(wb2, sw_hbm.at[p2i], sem)
    a1.wait()
    a2.wait()
    a3.wait()
    a4.wait()


def _combine_body(y_hbm, sh_hbm, meta_hbm, out_hbm,
                  r1, r2, shv, ob, mrows, p1i, p2i, sem):
    wid = lax.axis_index("s") * NC + lax.axis_index("c")
    base = wid * TOKW

    for r in range(2):
        pltpu.sync_copy(meta_hbm.at[r, pl.ds(base, TOKW)], mrows.at[r])
    for k in range(TOKW // CHUNK):
        sl = pl.ds(k * CHUNK, CHUNK)
        p1i[sl] = mrows[0, sl].astype(jnp.int32)
        p2i[sl] = mrows[1, sl].astype(jnp.int32)

    def chunk(c, _):
        tb = base + c * CHUNK
        csl = pl.ds(c * CHUNK, CHUNK)
        a1 = pltpu.async_copy(y_hbm.at[p1i.at[csl]], r1, sem)
        a2 = pltpu.async_copy(y_hbm.at[p2i.at[csl]], r2, sem)
        a3 = pltpu.async_copy(sh_hbm.at[pl.ds(tb, CHUNK)], shv, sem)
        a1.wait()
        a2.wait()
        a3.wait()

        def row(i, _):
            for v in range(D // 16):
                sv = pl.ds(v * 16, 16)
                ob[i, sv] = r1[i, sv] + r2[i, sv] + shv[i, sv]
            return 0

        lax.fori_loop(0, CHUNK, row, 0)
        pltpu.sync_copy(ob, out_hbm.at[pl.ds(tb, CHUNK)])
        return 0

    lax.fori_loop(0, TOKW // CHUNK, chunk, 0)


def _make_sc_kernels():
    mesh = plsc.VectorSubcoreMesh(core_axis_name="c", subcore_axis_name="s",
                                  num_cores=NC, num_subcores=NS)
    dispatch = pl.kernel(
        _dispatch_body, mesh=mesh,
        out_type=[jax.ShapeDtypeStruct((P, D), jnp.float32),
                  jax.ShapeDtypeStruct((P,), jnp.float32)],
        scratch_types=[
            pltpu.VMEM((TOKW, D), jnp.float32),
            pltpu.VMEM((4, TOKW), jnp.float32),
            pltpu.VMEM((TOKW,), jnp.int32),
            pltpu.VMEM((TOKW,), jnp.int32),
            pltpu.VMEM((TOKW,), jnp.float32),
            pltpu.VMEM((TOKW,), jnp.float32),
            pltpu.SemaphoreType.DMA,
        ])
    combine = pl.kernel(
        _combine_body, mesh=mesh,
        out_type=jax.ShapeDtypeStruct((T, D), jnp.float32),
        scratch_types=[
            pltpu.VMEM((CHUNK, D), jnp.float32),
            pltpu.VMEM((CHUNK, D), jnp.float32),
            pltpu.VMEM((CHUNK, D), jnp.float32),
            pltpu.VMEM((CHUNK, D), jnp.float32),
            pltpu.VMEM((2, TOKW), jnp.float32),
            pltpu.VMEM((TOKW,), jnp.int32),
            pltpu.VMEM((TOKW,), jnp.int32),
            pltpu.SemaphoreType.DMA,
        ])
    return dispatch, combine


@jax.jit
def kernel(hidden_states, gate_w, e_bias, w_gate_up, w_down, ws_gate_up,
           ws_down):
    x = hidden_states.reshape(T, D)

    meta, bearr = pl.pallas_call(
        _router_body,
        out_shape=[jax.ShapeDtypeStruct((8, T), jnp.float32),
                   jax.ShapeDtypeStruct((8, 128), jnp.int32)],
    )(x, gate_w, e_bias.reshape(1, E))
    be = bearr[0, :NB]

    _dispatch, _combine = _make_sc_kernels()
    xs, sw = _dispatch(meta, x)

    sh = pl.pallas_call(
        _shared_body,
        out_shape=jax.ShapeDtypeStruct((T, D), jnp.float32),
    )(x, ws_gate_up, ws_down)

    y = pl.pallas_call(
        _routed_body,
        grid_spec=pltpu.PrefetchScalarGridSpec(
            num_scalar_prefetch=1,
            grid=(NB,),
            in_specs=[
                pl.BlockSpec((BS, D), lambda b, be: (b, 0)),
                pl.BlockSpec((BS, 1), lambda b, be: (b, 0)),
                pl.BlockSpec((1, D, 2 * F), lambda b, be: (be[b], 0, 0)),
                pl.BlockSpec((1, F, D), lambda b, be: (be[b], 0, 0)),
            ],
            out_specs=pl.BlockSpec((BS, D), lambda b, be: (b, 0)),
        ),
        out_shape=jax.ShapeDtypeStruct((P, D), jnp.float32),
    )(be, xs, sw.reshape(P, 1), w_gate_up, w_down)

    out = _combine(y, sh, meta)
    return out


# speedup vs baseline: 1.0743x; 1.0413x over previous
import functools
import jax
import jax.numpy as jnp
from jax import lax
from jax.experimental import pallas as pl
from jax.experimental.pallas import tpu as pltpu
from jax.experimental.pallas import tpu_sc as plsc

T = 2048
D = 1024
E = 8
F = 512
NG = 2
NSHARED = 2
SCALE = 2.5
FS = F * NSHARED

BS = 256
NB = (T * 2 - E) // BS + E
P = NB * BS
RC = 256

NC = 2
NS = 16
NW = NC * NS
TOKW = T // NW
CHUNK = 16


def _lane_cumsum(a):
    cols = [a[:, 0:1]]
    for j in range(1, a.shape[1]):
        cols.append(cols[-1] + a[:, j:j + 1])
    return jnp.concatenate(cols, axis=1)


def _sub_cumsum(a):
    rows = [a[0:1]]
    for j in range(1, a.shape[0]):
        rows.append(rows[-1] + a[j:j + 1])
    return jnp.concatenate(rows, axis=0)


def _router_body(x_ref, gw_ref, eb_ref, meta_ref, be_ref):
    x = x_ref[...]
    logits = jnp.dot(x, gw_ref[...])
    scores = jax.nn.sigmoid(logits)
    sfc = scores + eb_ref[...]

    def top2sum(grp):
        g1 = jnp.max(grp, axis=-1, keepdims=True)
        eq1 = jnp.where(grp == g1, 1.0, 0.0)
        first1 = (eq1 * _lane_cumsum(eq1)) == 1.0
        g2 = jnp.max(jnp.where(first1, -jnp.inf, grp), axis=-1, keepdims=True)
        return g1 + g2

    gs0 = top2sum(sfc[:, :E // NG])
    gs1 = top2sum(sfc[:, E // NG:])
    gsel_f = jnp.where(gs0 >= gs1, 1.0, 0.0)
    gof = (jax.lax.broadcasted_iota(jnp.int32, (T, E), 1) // (E // NG)
           ).astype(jnp.float32)
    emask_f = gsel_f * (1.0 - gof) + (1.0 - gsel_f) * gof

    masked = jnp.where(emask_f > 0.5, sfc, -1e9)
    m1 = jnp.max(masked, axis=-1, keepdims=True)
    e1 = jnp.where(masked == m1, 1.0, 0.0)
    f1 = (e1 * _lane_cumsum(e1)) == 1.0
    masked2 = jnp.where(f1, -jnp.inf, masked)
    m2 = jnp.max(masked2, axis=-1, keepdims=True)
    e2 = jnp.where(masked2 == m2, 1.0, 0.0)
    f2 = (e2 * _lane_cumsum(e2)) == 1.0

    w1 = jnp.sum(jnp.where(f1, scores, 0.0), axis=-1, keepdims=True)
    w2 = jnp.sum(jnp.where(f2, scores, 0.0), axis=-1, keepdims=True)
    inv = SCALE / (w1 + w2 + 1e-20)
    w1s = w1 * inv
    w2s = w2 * inv

    sel = jnp.where(f1, 1.0, 0.0) + jnp.where(f2, 1.0, 0.0)
    tri = jnp.where(
        jax.lax.broadcasted_iota(jnp.int32, (RC, RC), 0)
        > jax.lax.broadcasted_iota(jnp.int32, (RC, RC), 1), 1.0, 0.0)
    ranks = []
    carry = jnp.zeros((1, E), jnp.float32)
    for i in range(T // RC):
        ch = sel[i * RC:(i + 1) * RC]
        rloc = jnp.dot(tri, ch, preferred_element_type=jnp.float32)
        ranks.append(rloc + carry)
        carry = carry + jnp.sum(ch, axis=0, keepdims=True)
    rank = jnp.concatenate(ranks, axis=0)

    counts_row = carry
    nb_row = jnp.floor(counts_row * (1.0 / BS) + (BS - 1) / BS)
    start_row = (_lane_cumsum(nb_row) - nb_row) * BS
    pos = start_row + rank
    pos1 = jnp.sum(jnp.where(f1, pos, 0.0), axis=-1, keepdims=True)
    pos2 = jnp.sum(jnp.where(f2, pos, 0.0), axis=-1, keepdims=True)

    meta4 = jnp.concatenate([pos1, pos2, w1s, w2s], axis=1)
    meta_ref[...] = jnp.concatenate(
        [lax.transpose(meta4, (1, 0)),
         jnp.zeros((4, T), jnp.float32)], axis=0)

    counts_col = lax.dot_general(sel, jnp.ones((T, 1), jnp.float32),
                                 (((0,), (0,)), ((), ())))
    nb_col = jnp.floor(counts_col * (1.0 / BS) + (BS - 1) / BS)
    end_col = _sub_cumsum(nb_col)
    b_io = jax.lax.broadcasted_iota(jnp.int32, (E, 128), 1).astype(jnp.float32)
    cmp = jnp.where(b_io >= end_col, 1.0, 0.0)
    be_row = jnp.minimum(jnp.sum(cmp, axis=0, keepdims=True), E - 1)
    sub0 = jax.lax.broadcasted_iota(jnp.int32, (8, 128), 0) == 0
    be_ref[...] = jnp.where(sub0, be_row, 0.0).astype(jnp.int32)


def _shared_body(x_ref, wsgu_ref, wsdn_ref, sh_ref):
    x = x_ref[...]
    for hh in range(NSHARED):
        g = jnp.dot(x, wsgu_ref[:, hh * F:(hh + 1) * F],
                    preferred_element_type=jnp.float32)
        u = jnp.dot(x, wsgu_ref[:, FS + hh * F:FS + (hh + 1) * F],
                    preferred_element_type=jnp.float32)
        hq = g * jax.nn.sigmoid(g) * u
        y = jnp.dot(hq, wsdn_ref[hh * F:(hh + 1) * F, :],
                    preferred_element_type=jnp.float32)
        if hh == 0:
            sh_ref[...] = y
        else:
            sh_ref[...] += y


def _routed_body(be_ref, xs_ref, sw_ref, wgu_ref, wdn_ref, y_ref):
    del be_ref
    xb = xs_ref[...]
    gu = jnp.dot(xb, wgu_ref[0], preferred_element_type=jnp.float32)
    g = gu[:, :F]
    h = g * jax.nn.sigmoid(g) * gu[:, F:]
    y = jnp.dot(h, wdn_ref[0], preferred_element_type=jnp.float32)
    y_ref[...] = y * sw_ref[...]


def _dispatch_body(meta_hbm, x_hbm, xs_hbm, sw_hbm,
                   xrows, mrows, p1i, p2i, wb1, wb2, sem):
    wid = lax.axis_index("s") * NC + lax.axis_index("c")
    base = wid * TOKW

    for r in range(4):
        pltpu.sync_copy(meta_hbm.at[r, pl.ds(base, TOKW)], mrows.at[r])
    pltpu.sync_copy(x_hbm.at[pl.ds(base, TOKW)], xrows)
    for k in range(TOKW // CHUNK):
        sl = pl.ds(k * CHUNK, CHUNK)
        p1i[sl] = mrows[0, sl].astype(jnp.int32)
        p2i[sl] = mrows[1, sl].astype(jnp.int32)
        wb1[sl] = mrows[2, sl]
        wb2[sl] = mrows[3, sl]
    a1 = pltpu.async_copy(xrows, xs_hbm.at[p1i], sem)
    a2 = pltpu.async_copy(xrows, xs_hbm.at[p2i], sem)
    a3 = pltpu.async_copy(wb1, sw_hbm.at[p1i], sem)
    a4 = pltpu.async_copy(wb2, sw_hbm.at[p2i], sem)
    a1.wait()
    a2.wait()
    a3.wait()
    a4.wait()


def _combine_body(y_hbm, sh_hbm, meta_hbm, out_hbm,
                  r1, r2, shv, ob, mrows, p1i, p2i, sem):
    wid = lax.axis_index("s") * NC + lax.axis_index("c")
    base = wid * TOKW

    for r in range(2):
        pltpu.sync_copy(meta_hbm.at[r, pl.ds(base, TOKW)], mrows.at[r])
    for k in range(TOKW // CHUNK):
        sl = pl.ds(k * CHUNK, CHUNK)
        p1i[sl] = mrows[0, sl].astype(jnp.int32)
        p2i[sl] = mrows[1, sl].astype(jnp.int32)

    def chunk(c, _):
        tb = base + c * CHUNK
        csl = pl.ds(c * CHUNK, CHUNK)
        a1 = pltpu.async_copy(y_hbm.at[p1i.at[csl]], r1, sem)
        a2 = pltpu.async_copy(y_hbm.at[p2i.at[csl]], r2, sem)
        a3 = pltpu.async_copy(sh_hbm.at[pl.ds(tb, CHUNK)], shv, sem)
        a1.wait()
        a2.wait()
        a3.wait()

        def row(i, _):
            for v in range(D // 16):
                sv = pl.ds(v * 16, 16)
                ob[i, sv] = r1[i, sv] + r2[i, sv] + shv[i, sv]
            return 0

        lax.fori_loop(0, CHUNK, row, 0)
        pltpu.sync_copy(ob, out_hbm.at[pl.ds(tb, CHUNK)])
        return 0

    lax.fori_loop(0, TOKW // CHUNK, chunk, 0)


def _make_sc_kernels():
    mesh = plsc.VectorSubcoreMesh(core_axis_name="c", subcore_axis_name="s",
                                  num_cores=NC, num_subcores=NS)
    dispatch = pl.kernel(
        _dispatch_body, mesh=mesh,
        out_type=[jax.ShapeDtypeStruct((P, D), jnp.float32),
                  jax.ShapeDtypeStruct((P,), jnp.float32)],
        scratch_types=[
            pltpu.VMEM((TOKW, D), jnp.float32),
            pltpu.VMEM((4, TOKW), jnp.float32),
            pltpu.VMEM((TOKW,), jnp.int32),
            pltpu.VMEM((TOKW,), jnp.int32),
            pltpu.VMEM((TOKW,), jnp.float32),
            pltpu.VMEM((TOKW,), jnp.float32),
            pltpu.SemaphoreType.DMA,
        ])
    combine = pl.kernel(
        _combine_body, mesh=mesh,
        out_type=jax.ShapeDtypeStruct((T, D), jnp.float32),
        scratch_types=[
            pltpu.VMEM((CHUNK, D), jnp.float32),
            pltpu.VMEM((CHUNK, D), jnp.float32),
            pltpu.VMEM((CHUNK, D), jnp.float32),
            pltpu.VMEM((CHUNK, D), jnp.float32),
            pltpu.VMEM((2, TOKW), jnp.float32),
            pltpu.VMEM((TOKW,), jnp.int32),
            pltpu.VMEM((TOKW,), jnp.int32),
            pltpu.SemaphoreType.DMA,
        ])
    return dispatch, combine


@jax.jit
def kernel(hidden_states, gate_w, e_bias, w_gate_up, w_down, ws_gate_up,
           ws_down):
    x = hidden_states.reshape(T, D)

    meta, bearr = pl.pallas_call(
        _router_body,
        out_shape=[jax.ShapeDtypeStruct((8, T), jnp.float32),
                   jax.ShapeDtypeStruct((8, 128), jnp.int32)],
    )(x, gate_w, e_bias.reshape(1, E))
    be = bearr[0, :NB]

    _dispatch, _combine = _make_sc_kernels()
    xs, sw = _dispatch(meta, x)

    sh = pl.pallas_call(
        _shared_body,
        out_shape=jax.ShapeDtypeStruct((T, D), jnp.float32),
    )(x, ws_gate_up, ws_down)

    y = pl.pallas_call(
        _routed_body,
        grid_spec=pltpu.PrefetchScalarGridSpec(
            num_scalar_prefetch=1,
            grid=(NB,),
            in_specs=[
                pl.BlockSpec((BS, D), lambda b, be: (b, 0)),
                pl.BlockSpec((BS, 1), lambda b, be: (b, 0)),
                pl.BlockSpec((1, D, 2 * F), lambda b, be: (be[b], 0, 0)),
                pl.BlockSpec((1, F, D), lambda b, be: (be[b], 0, 0)),
            ],
            out_specs=pl.BlockSpec((BS, D), lambda b, be: (b, 0)),
        ),
        out_shape=jax.ShapeDtypeStruct((P, D), jnp.float32),
    )(be, xs, sw.reshape(P, 1), w_gate_up, w_down)

    out = _combine(y, sh, meta)
    return out
